# baseline (speedup 1.0000x reference)
import functools

import jax
import jax.numpy as jnp
from jax import lax
from jax.experimental import pallas as pl
from jax.experimental.pallas import tpu as pltpu
from jax.experimental.pallas import tpu_sc as plsc

_IN_H = 512
_IN_W = 512
_HW = _IN_H * _IN_W
_O = _HW
_K = 16
_B = 64
_L = 16

_NC = 2
_NS = 16
_NW = _NC * _NS
_OPW = _O // _NW
_T = 64
_STEPS = _OPW // _T
_R = _T * _K
_NG = _R // 128


def _sc_body(table, conn, w, out, connbufs, cflats, wbufs, rowsbufs, outbufs,
             gsems, csems, wsems, osems):
    wid = lax.axis_index("s") * _NC + lax.axis_index("c")
    o0 = wid * _OPW

    def flatten_conn(par):
        for k in range(_K):
            for c in range(_T // _L):
                cflats[par][pl.ds(k * _T + c * _L, _L)] = \
                    connbufs[par][k, pl.ds(c * _L, _L)]

    def issue_gathers(par):
        for j in range(_NG):
            pltpu.async_copy(
                table.at[cflats[par].at[pl.ds(j * 128, 128)]],
                rowsbufs[par].at[pl.ds(j * 128, 128)],
                gsems[par],
            )

    def start_conn(s, par):
        pltpu.async_copy(conn.at[:, pl.ds(o0 + s * _T, _T)],
                         connbufs[par], csems[par])

    def start_w(s, par):
        pltpu.async_copy(w.at[:, pl.ds(o0 + s * _T, _T)],
                         wbufs[par], wsems[par])

    def wait_conn(par):
        pltpu.make_async_copy(conn.at[:, pl.ds(0, _T)], connbufs[par],
                              csems[par]).wait()

    def wait_w(par):
        pltpu.make_async_copy(w.at[:, pl.ds(0, _T)], wbufs[par],
                              wsems[par]).wait()

    def wait_gathers(par):
        pltpu.make_async_copy(table.at[pl.ds(0, _R)], rowsbufs[par],
                              gsems[par]).wait()

    def wait_out(par):
        pltpu.make_async_copy(out.at[:, pl.ds(0, _T)], outbufs[par],
                              osems[par]).wait()

    pltpu.sync_copy(conn.at[:, pl.ds(o0, _T)], connbufs[0])
    flatten_conn(0)
    issue_gathers(0)
    start_conn(1, 1)
    start_w(0, 0)
    start_w(1, 1)

    kiota = lax.iota(jnp.int32, _L)
    eidx = [lax.iota(jnp.int32, _L) * 2 + j * 32 for j in range(_B // 32)]
    oidx = [lax.iota(jnp.int32, _L) * 2 + j * 32 + 1 for j in range(_B // 32)]

    def pair(s2, carry):
        for par in range(2):
            s = s2 * 2 + par
            parn = par ^ 1
            ob = o0 + s * _T

            wait_gathers(par)

            @pl.when(s + 2 < _STEPS)
            def _():
                start_conn(s + 2, par)

            @pl.when(s + 1 < _STEPS)
            def _():
                wait_conn(parn)
                flatten_conn(parn)
                issue_gathers(parn)

            wait_w(par)

            @pl.when(s >= 2)
            def _():
                wait_out(par)

            def emit_out(o):
                wrow = plsc.load_gather(
                    wbufs[par], [kiota, jnp.full((_L,), o, jnp.int32)])
                col = jnp.full((_L,), o, jnp.int32)
                accs = [[None] * 4 for _ in range(_B // 32)]
                for k in range(_K):
                    wk = lax.gather(
                        wrow, jnp.full((_L, 1), k, jnp.int32),
                        lax.GatherDimensionNumbers(
                            offset_dims=(), collapsed_slice_dims=(0,),
                            start_index_map=(0,)),
                        slice_sizes=(1,),
                        mode=lax.GatherScatterMode.PROMISE_IN_BOUNDS)
                    wkb = plsc.pack(wk, wk,
                                    format=plsc.PackFormat.INTERLEAVED)
                    g = k // 4
                    for j in range(_B // 32):
                        x = rowsbufs[par][k * _T + o, pl.ds(j * 32, 32)]
                        p = x * wkb
                        accs[j][g] = p if accs[j][g] is None \
                            else accs[j][g] + p
                for j in range(_B // 32):
                    a = accs[j]
                    t = (a[0] + a[1]) + (a[2] + a[3])
                    xe, xo = plsc.unpack(
                        t, format=plsc.PackFormat.INTERLEAVED)
                    plsc.store_scatter(outbufs[par], [eidx[j], col], xe)
                    plsc.store_scatter(outbufs[par], [oidx[j], col], xo)

            def four_out(i, carry2):
                for u in range(4):
                    emit_out(i * 4 + u)
                return carry2

            lax.fori_loop(0, _T // 4, four_out, 0)

            @pl.when(s + 2 < _STEPS)
            def _():
                start_w(s + 2, par)

            pltpu.async_copy(outbufs[par], out.at[:, pl.ds(ob, _T)],
                             osems[par])
        return carry

    lax.fori_loop(0, _STEPS // 2, pair, 0)
    wait_out(0)
    wait_out(1)


def kernel(input, connections, weights):
    table = input.reshape(_B, _HW).T.astype(jnp.bfloat16)

    mesh = plsc.VectorSubcoreMesh(core_axis_name="c", subcore_axis_name="s")
    fn = functools.partial(
        pl.kernel,
        mesh=mesh,
        out_type=jax.ShapeDtypeStruct((_B, _O), jnp.float32),
        scratch_types=[
            [pltpu.VMEM((_K, _T), jnp.int32) for _ in range(2)],
            [pltpu.VMEM((_R,), jnp.int32) for _ in range(2)],
            [pltpu.VMEM((_K, _T), jnp.float32) for _ in range(2)],
            [pltpu.VMEM((_R, _B), jnp.bfloat16) for _ in range(2)],
            [pltpu.VMEM((_B, _T), jnp.float32) for _ in range(2)],
            [pltpu.SemaphoreType.DMA for _ in range(2)],
            [pltpu.SemaphoreType.DMA for _ in range(2)],
            [pltpu.SemaphoreType.DMA for _ in range(2)],
            [pltpu.SemaphoreType.DMA for _ in range(2)],
        ],
        compiler_params=pltpu.CompilerParams(
            use_tc_tiling_on_sc=False,
            needs_layout_passes=False,
        ),
    )(_sc_body)
    out = fn(table, connections.T, weights.T)
    return out.reshape(_B, _IN_H, _IN_W)
